# baseline (speedup 1.0000x reference)
import functools

import jax
import jax.numpy as jnp
from jax import lax
from jax.experimental import pallas as pl
from jax.experimental.pallas import tpu as pltpu
from jax.experimental.pallas import tpu_sc as plsc

N_I = 16384
N_J = 50
D = 32

NC = 2
NS = 16
NW = NC * NS
I_PER_W = N_I // NW
NT = I_PER_W // 128
NBUF = 2 * NT

_mesh = plsc.VectorSubcoreMesh(core_axis_name="c", subcore_axis_name="s")

N_EMB_ROWS = 1000000
N_UNITS = N_EMB_ROWS // 128
N_FULL = N_UNITS - N_UNITS % NW
NBA = 4


@functools.partial(
    pl.kernel,
    out_type=jax.ShapeDtypeStruct((N_EMB_ROWS * D,), jnp.float32),
    mesh=_mesh,
    scratch_types=[
        pltpu.VMEM((D, 128), jnp.float32),
        pltpu.VMEM((D, 128), jnp.float32),
        pltpu.VMEM((D, 128), jnp.float32),
        pltpu.VMEM((D, 128), jnp.float32),
        pltpu.VMEM((D * 128,), jnp.float32),
        pltpu.VMEM((D * 128,), jnp.float32),
        pltpu.VMEM((D * 128,), jnp.float32),
        pltpu.VMEM((D * 128,), jnp.float32),
        pltpu.VMEM((D, 64), jnp.float32),
        pltpu.VMEM((D * 64,), jnp.float32),
        pltpu.SemaphoreType.DMA((NBA,)),
        pltpu.SemaphoreType.DMA((NBA,)),
        pltpu.SemaphoreType.DMA,
    ],
    compiler_params=pltpu.CompilerParams(
        use_tc_tiling_on_sc=True, needs_layout_passes=False,
        disable_bounds_checks=True),
)
def _sc_repack(embT_hbm, wide_hbm, ibuf0, ibuf1, ibuf2, ibuf3,
               obuf0, obuf1, obuf2, obuf3, tibuf, tobuf, gsem, wsem, tsem):
    ibufs = [ibuf0, ibuf1, ibuf2, ibuf3]
    obufs = [obuf0, obuf1, obuf2, obuf3]
    wid = lax.axis_index("s") * NC + lax.axis_index("c")
    iota = lax.iota(jnp.int32, 16)

    def fire_in(u, b):
        pltpu.async_copy(
            embT_hbm.at[:, pl.ds(u * 128, 128)], ibufs[b], gsem.at[b])

    def drain_in(u, b):
        pltpu.make_async_copy(
            embT_hbm.at[:, pl.ds(u * 128, 128)], ibufs[b], gsem.at[b]).wait()

    colv = [16 * L + iota for L in range(8)]
    skc = (iota // 4) * 128 + (iota % 4) * 32

    def transpose_unit(src, dst, nl):
        def kbody(k, carry):
            for k2 in (k, k + 8):
                dk = (iota + k2) & 15
                dk16 = dk + 16
                sk = skc + dk
                for L in range(nl):
                    for h in range(2):
                        v = plsc.load_gather(
                            src, [dk if h == 0 else dk16, colv[L]])
                        plsc.store_scatter(dst, [sk + (512 * L + 16 * h)], v)
            return carry
        lax.fori_loop(0, 8, kbody, 0)

    def fire_out(u, b):
        pltpu.async_copy(
            obufs[b], wide_hbm.at[pl.ds(u * 4096, 4096)], wsem.at[b])

    def drain_out(u, b):
        pltpu.make_async_copy(
            obufs[b], wide_hbm.at[pl.ds(u * 4096, 4096)], wsem.at[b]).wait()

    for s in range(NBA):
        fire_in(wid + NW * s, s)

    n_g = N_FULL // (NBA * NW)

    def gbody(g, carry):
        for s in range(NBA):
            u = wid + NW * (NBA * g + s)
            drain_in(u, s)

            @pl.when(g > 0)
            def _():
                drain_out(u - NBA * NW, s)
            transpose_unit(ibufs[s], obufs[s], 8)
            fire_out(u, s)

            @pl.when(g < n_g - 1)
            def _():
                fire_in(u + NBA * NW, s)
        return carry

    lax.fori_loop(0, n_g, gbody, 0)
    for s in range(NBA):
        drain_out(N_FULL - NBA * NW + wid + NW * s, s)

    @pl.when(wid < 4)
    def _():
        u = N_FULL + wid
        pltpu.sync_copy(embT_hbm.at[:, pl.ds(u * 128, 128)], ibufs[0])
        transpose_unit(ibufs[0], obufs[0], 8)
        pltpu.async_copy(
            obufs[0], wide_hbm.at[pl.ds(u * 4096, 4096)], wsem.at[0]).wait()

    @pl.when(wid == 4)
    def _():
        c0 = N_UNITS * 128
        pltpu.sync_copy(embT_hbm.at[:, pl.ds(c0, 64)], tibuf)
        transpose_unit(tibuf, tobuf, 4)
        pltpu.async_copy(
            tobuf, wide_hbm.at[pl.ds(c0 * D, 64 * D)], tsem).wait()


@functools.partial(
    pl.kernel,
    out_type=jax.ShapeDtypeStruct((N_J, D // 8, N_I // 128, 8 * 128), jnp.float32),
    mesh=_mesh,
    scratch_types=[
        pltpu.VMEM((N_J, I_PER_W), jnp.int32),
        pltpu.VMEM((NBUF, 128, D), jnp.float32),
        pltpu.VMEM((NBUF, D * 128), jnp.float32),
        pltpu.SemaphoreType.DMA((NBUF,)),
        pltpu.SemaphoreType.DMA((NBUF,)),
    ],
    compiler_params=pltpu.CompilerParams(
        use_tc_tiling_on_sc=False, needs_layout_passes=False,
        disable_bounds_checks=True),
)
def _sc_gather(dataT_hbm, emb_hbm, out5_hbm, idxT_v, gbufs, lbufs, gsem, wsem):
    wid = lax.axis_index("s") * NC + lax.axis_index("c")
    i0 = wid * I_PER_W

    pltpu.sync_copy(dataT_hbm.at[:, pl.ds(i0, I_PER_W)], idxT_v)

    iota = lax.iota(jnp.int32, 16)
    rowv = [16 * L + iota for L in range(8)]

    def fire_gathers(j, base):
        for t in range(NT):
            pltpu.async_copy(
                emb_hbm.at[idxT_v.at[j, pl.ds(t * 128, 128)]],
                gbufs.at[base + t], gsem.at[base + t])

    def drain_gathers(j, base):
        for t in range(NT):
            pltpu.make_async_copy(
                emb_hbm.at[idxT_v.at[j, pl.ds(t * 128, 128)]],
                gbufs.at[base + t], gsem.at[base + t]).wait()

    def drain_writes(j, base):
        for t in range(NT):
            for tc in range(D // 8):
                pltpu.make_async_copy(
                    lbufs.at[base + t, pl.ds(tc * 1024, 1024)],
                    out5_hbm.at[j, tc, wid * NT + t],
                    wsem.at[base + t]).wait()

    def transpose_and_write(j, base):
        for t in range(NT):
            def kbody(k, carry):
                for k2 in (k, k + 8):
                    dk = (iota + k2) & 15
                    dk16 = dk + 16
                    sk = dk * 128 + iota
                    for L in range(8):
                        for h in range(2):
                            v = plsc.load_gather(
                                gbufs.at[base + t],
                                [rowv[L], dk if h == 0 else dk16])
                            plsc.store_scatter(
                                lbufs.at[base + t],
                                [sk + (2048 * h + 16 * L)], v)
                return carry
            lax.fori_loop(0, 8, kbody, 0)
            for tc in range(D // 8):
                pltpu.async_copy(
                    lbufs.at[base + t, pl.ds(tc * 1024, 1024)],
                    out5_hbm.at[j, tc, wid * NT + t],
                    wsem.at[base + t])

    fire_gathers(0, 0)

    def gbody(g, carry):
        j0 = 2 * g
        fire_gathers(j0 + 1, NT)

        drain_gathers(j0, 0)

        @pl.when(g > 0)
        def _():
            drain_writes(j0 - 2, 0)
        transpose_and_write(j0, 0)

        @pl.when(g < (N_J // 2 - 1))
        def _():
            fire_gathers(j0 + 2, 0)

        drain_gathers(j0 + 1, NT)

        @pl.when(g > 0)
        def _():
            drain_writes(j0 - 1, NT)
        transpose_and_write(j0 + 1, NT)
        return carry

    lax.fori_loop(0, N_J // 2, gbody, 0)

    drain_writes(N_J - 2, 0)
    drain_writes(N_J - 1, NT)


def kernel(data, emb):
    wide = _sc_repack(emb.T)
    out5 = _sc_gather(data.T.astype(jnp.int32), wide.reshape(N_EMB_ROWS, D))
    out5 = out5.reshape(N_J, D // 8, N_I // 128, 8, 128)
    return out5.transpose(2, 4, 0, 1, 3).reshape(N_I, N_J, D)


# speedup vs baseline: 1.0123x; 1.0123x over previous
import functools

import jax
import jax.numpy as jnp
from jax import lax
from jax.experimental import pallas as pl
from jax.experimental.pallas import tpu as pltpu
from jax.experimental.pallas import tpu_sc as plsc

N_I = 16384
N_J = 50
D = 32

NC = 2
NS = 16
NW = NC * NS
I_PER_W = N_I // NW
NT = I_PER_W // 128
NBUF = 2 * NT

_mesh = plsc.VectorSubcoreMesh(core_axis_name="c", subcore_axis_name="s")

N_EMB_ROWS = 1000000
N_UNITS = N_EMB_ROWS // 128
N_FULL = N_UNITS - N_UNITS % NW
NBA = 4


@functools.partial(
    pl.kernel,
    out_type=jax.ShapeDtypeStruct((N_EMB_ROWS * D,), jnp.float32),
    mesh=_mesh,
    scratch_types=[
        pltpu.VMEM((D, 128), jnp.float32),
        pltpu.VMEM((D, 128), jnp.float32),
        pltpu.VMEM((D, 128), jnp.float32),
        pltpu.VMEM((D, 128), jnp.float32),
        pltpu.VMEM((D * 128,), jnp.float32),
        pltpu.VMEM((D * 128,), jnp.float32),
        pltpu.VMEM((D * 128,), jnp.float32),
        pltpu.VMEM((D * 128,), jnp.float32),
        pltpu.VMEM((D, 64), jnp.float32),
        pltpu.VMEM((D * 64,), jnp.float32),
        pltpu.SemaphoreType.DMA((NBA,)),
        pltpu.SemaphoreType.DMA((NBA,)),
        pltpu.SemaphoreType.DMA,
    ],
    compiler_params=pltpu.CompilerParams(
        use_tc_tiling_on_sc=True, needs_layout_passes=False,
        disable_bounds_checks=True),
)
def _sc_repack(embT_hbm, wide_hbm, ibuf0, ibuf1, ibuf2, ibuf3,
               obuf0, obuf1, obuf2, obuf3, tibuf, tobuf, gsem, wsem, tsem):
    ibufs = [ibuf0, ibuf1, ibuf2, ibuf3]
    obufs = [obuf0, obuf1, obuf2, obuf3]
    wid = lax.axis_index("s") * NC + lax.axis_index("c")
    iota = lax.iota(jnp.int32, 16)

    def fire_in(u, b):
        pltpu.async_copy(
            embT_hbm.at[:, pl.ds(u * 128, 128)], ibufs[b], gsem.at[b])

    def drain_in(u, b):
        pltpu.make_async_copy(
            embT_hbm.at[:, pl.ds(u * 128, 128)], ibufs[b], gsem.at[b]).wait()

    colv = [16 * L + iota for L in range(8)]
    skc = (iota // 4) * 128 + (iota % 4) * 32

    def transpose_unit(src, dst, nl):
        def kbody(k, carry):
            dk = (iota + k) & 15
            dk16 = dk + 16
            sk = skc + dk
            for L in range(nl):
                for h in range(2):
                    v = plsc.load_gather(src, [dk if h == 0 else dk16, colv[L]])
                    plsc.store_scatter(dst, [sk + (512 * L + 16 * h)], v)
            return carry
        lax.fori_loop(0, 16, kbody, 0)

    def fire_out(u, b):
        pltpu.async_copy(
            obufs[b], wide_hbm.at[pl.ds(u * 4096, 4096)], wsem.at[b])

    def drain_out(u, b):
        pltpu.make_async_copy(
            obufs[b], wide_hbm.at[pl.ds(u * 4096, 4096)], wsem.at[b]).wait()

    for s in range(NBA):
        fire_in(wid + NW * s, s)

    n_g = N_FULL // (NBA * NW)

    def gbody(g, carry):
        for s in range(NBA):
            u = wid + NW * (NBA * g + s)
            drain_in(u, s)

            @pl.when(g > 0)
            def _():
                drain_out(u - NBA * NW, s)
            transpose_unit(ibufs[s], obufs[s], 8)
            fire_out(u, s)

            @pl.when(g < n_g - 1)
            def _():
                fire_in(u + NBA * NW, s)
        return carry

    lax.fori_loop(0, n_g, gbody, 0)
    for s in range(NBA):
        drain_out(N_FULL - NBA * NW + wid + NW * s, s)

    @pl.when(wid < 4)
    def _():
        u = N_FULL + wid
        pltpu.sync_copy(embT_hbm.at[:, pl.ds(u * 128, 128)], ibufs[0])
        transpose_unit(ibufs[0], obufs[0], 8)
        pltpu.async_copy(
            obufs[0], wide_hbm.at[pl.ds(u * 4096, 4096)], wsem.at[0]).wait()

    @pl.when(wid == 4)
    def _():
        c0 = N_UNITS * 128
        pltpu.sync_copy(embT_hbm.at[:, pl.ds(c0, 64)], tibuf)
        transpose_unit(tibuf, tobuf, 4)
        pltpu.async_copy(
            tobuf, wide_hbm.at[pl.ds(c0 * D, 64 * D)], tsem).wait()


@functools.partial(
    pl.kernel,
    out_type=jax.ShapeDtypeStruct((N_J, D // 8, N_I // 128, 8 * 128), jnp.float32),
    mesh=_mesh,
    scratch_types=[
        pltpu.VMEM((N_J, I_PER_W), jnp.int32),
        pltpu.VMEM((NBUF, 128, D), jnp.float32),
        pltpu.VMEM((NBUF, D * 128), jnp.float32),
        pltpu.SemaphoreType.DMA((NBUF,)),
        pltpu.SemaphoreType.DMA((NBUF,)),
    ],
    compiler_params=pltpu.CompilerParams(
        use_tc_tiling_on_sc=False, needs_layout_passes=False,
        disable_bounds_checks=True),
)
def _sc_gather(dataT_hbm, emb_hbm, out5_hbm, idxT_v, gbufs, lbufs, gsem, wsem):
    wid = lax.axis_index("s") * NC + lax.axis_index("c")
    i0 = wid * I_PER_W

    pltpu.sync_copy(dataT_hbm.at[:, pl.ds(i0, I_PER_W)], idxT_v)

    iota = lax.iota(jnp.int32, 16)
    rowv = [16 * L + iota for L in range(8)]

    def fire_gathers(j, base):
        for t in range(NT):
            pltpu.async_copy(
                emb_hbm.at[idxT_v.at[j, pl.ds(t * 128, 128)]],
                gbufs.at[base + t], gsem.at[base + t])

    def drain_gathers(j, base):
        for t in range(NT):
            pltpu.make_async_copy(
                emb_hbm.at[idxT_v.at[j, pl.ds(t * 128, 128)]],
                gbufs.at[base + t], gsem.at[base + t]).wait()

    def drain_writes(j, base):
        for t in range(NT):
            for tc in range(D // 8):
                pltpu.make_async_copy(
                    lbufs.at[base + t, pl.ds(tc * 1024, 1024)],
                    out5_hbm.at[j, tc, wid * NT + t],
                    wsem.at[base + t]).wait()

    def transpose_and_write(j, base):
        for t in range(NT):
            def kbody(k, carry):
                dk = (iota + k) & 15
                dk16 = dk + 16
                sk = dk * 128 + iota
                for L in range(8):
                    for h in range(2):
                        v = plsc.load_gather(
                            gbufs.at[base + t],
                            [rowv[L], dk if h == 0 else dk16])
                        plsc.store_scatter(
                            lbufs.at[base + t],
                            [sk + (2048 * h + 16 * L)], v)
                return carry
            lax.fori_loop(0, 16, kbody, 0)
            for tc in range(D // 8):
                pltpu.async_copy(
                    lbufs.at[base + t, pl.ds(tc * 1024, 1024)],
                    out5_hbm.at[j, tc, wid * NT + t],
                    wsem.at[base + t])

    fire_gathers(0, 0)

    def gbody(g, carry):
        j0 = 2 * g
        fire_gathers(j0 + 1, NT)

        drain_gathers(j0, 0)

        @pl.when(g > 0)
        def _():
            drain_writes(j0 - 2, 0)
        transpose_and_write(j0, 0)

        @pl.when(g < (N_J // 2 - 1))
        def _():
            fire_gathers(j0 + 2, 0)

        drain_gathers(j0 + 1, NT)

        @pl.when(g > 0)
        def _():
            drain_writes(j0 - 1, NT)
        transpose_and_write(j0 + 1, NT)
        return carry

    lax.fori_loop(0, N_J // 2, gbody, 0)

    drain_writes(N_J - 2, 0)
    drain_writes(N_J - 1, NT)


def kernel(data, emb):
    wide = _sc_repack(emb.T)
    out5 = _sc_gather(data.T.astype(jnp.int32), wide.reshape(N_EMB_ROWS, D))
    out5 = out5.reshape(N_J, D // 8, N_I // 128, 8, 128)
    return out5.transpose(2, 4, 0, 1, 3).reshape(N_I, N_J, D)
